# baseline (speedup 1.0000x reference)
import jax
import jax.numpy as jnp
from jax import lax
from jax.experimental import pallas as pl
from jax.experimental.pallas import tpu as pltpu

_N_ITER = 5
_G = 4


def _routing_body(u_ref, bexp_ref, v_ref, s1_ref):
    _, icaps, od = u_ref.shape
    g = _G
    d = 16
    u = u_ref[...]
    ko = lax.broadcasted_iota(jnp.int32, (od, od), 0) // d
    lo = lax.broadcasted_iota(jnp.int32, (od, od), 1) // d
    mask = (ko == lo).astype(jnp.bfloat16)

    b = jnp.broadcast_to(bexp_ref[...][None], (g, icaps, od))
    s1 = None
    vrow = None
    for r in range(_N_ITER):
        if r > 0:
            y = (u * vrow[:, None, :]).astype(jnp.bfloat16)
            z = lax.dot_general(
                y.reshape(g * icaps, od), mask,
                (((1,), (0,)), ((), ())),
                preferred_element_type=jnp.float32)
            b = b + z.reshape(g, icaps, od)
        m = jnp.max(b, axis=-1, keepdims=True)
        e = jnp.exp(b - m)
        ssum = jnp.sum(e, axis=-1, keepdims=True)
        c = e * (float(d) / ssum)
        s1 = c * u
        s2 = jnp.sum(s1, axis=1)
        p = s2 * s2
        p_hi = p.astype(jnp.bfloat16)
        p_lo = (p - p_hi.astype(jnp.float32)).astype(jnp.bfloat16)
        n2 = (lax.dot_general(p_hi, mask, (((1,), (0,)), ((), ())),
                              preferred_element_type=jnp.float32)
              + lax.dot_general(p_lo, mask, (((1,), (0,)), ((), ())),
                                preferred_element_type=jnp.float32))
        scale = n2 / (1.0 + n2) / jnp.sqrt(n2)
        vrow = s2 * scale
    v_ref[...] = vrow[None]
    s1_ref[...] = s1


def kernel(u_predict, b):
    bsz, icaps, ocaps, dim = u_predict.shape
    od = ocaps * dim
    u2 = u_predict.reshape(bsz, icaps, od)
    bexp0 = jnp.repeat(b, dim, axis=1)
    grid = (bsz // _G,)
    v, s1 = pl.pallas_call(
        _routing_body,
        grid=grid,
        in_specs=[
            pl.BlockSpec((_G, icaps, od), lambda i: (i, 0, 0)),
            pl.BlockSpec((icaps, od), lambda i: (0, 0)),
        ],
        out_specs=[
            pl.BlockSpec((1, _G, od), lambda i: (i, 0, 0)),
            pl.BlockSpec((_G, icaps, od), lambda i: (i, 0, 0)),
        ],
        out_shape=[
            jax.ShapeDtypeStruct((bsz // _G, _G, od), jnp.float32),
            jax.ShapeDtypeStruct((bsz, icaps, od), jnp.float32),
        ],
        compiler_params=pltpu.CompilerParams(
            dimension_semantics=("arbitrary",),
        ),
    )(u2, bexp0)
    return v.reshape(bsz, ocaps, dim), s1.reshape(bsz, icaps, ocaps, dim)


# speedup vs baseline: 3.0471x; 3.0471x over previous
import jax
import jax.numpy as jnp
from jax.experimental import pallas as pl
from jax.experimental.pallas import tpu as pltpu

_N_ITER = 5
_G = 4


def _routing_body(u_ref, bt_ref, v_ref, s1_ref):
    g = _G
    ocaps, icaps = bt_ref.shape
    od = u_ref.shape[1]
    dim = od // ocaps
    u = u_ref[...]
    u4 = u.reshape(g, ocaps, dim, icaps)

    b = jnp.broadcast_to(bt_ref[...][None], (g, ocaps, icaps))
    s1 = None
    vcol = None
    for r in range(_N_ITER):
        if r > 0:
            y4 = u4 * vcol.reshape(g, ocaps, dim, 1)
            b = b + jnp.sum(y4, axis=2)
        m = jnp.max(b, axis=1, keepdims=True)
        e = jnp.exp(b - m)
        c = e / jnp.sum(e, axis=1, keepdims=True)
        cexp = jnp.broadcast_to(c[:, :, None, :],
                                (g, ocaps, dim, icaps)).reshape(g, od, icaps)
        s1 = cexp * u
        s2 = jnp.sum(s1, axis=2, keepdims=True)
        s24 = s2.reshape(g, ocaps, dim, 1)
        n2 = jnp.sum(s24 * s24, axis=2, keepdims=True)
        scale = n2 / (1.0 + n2) / jnp.sqrt(n2)
        vcol = (s24 * scale).reshape(g, od, 1)
    v_ref[...] = vcol
    s1_ref[...] = s1


def kernel(u_predict, b):
    bsz, icaps, ocaps, dim = u_predict.shape
    od = ocaps * dim
    u_t = u_predict.transpose(0, 2, 3, 1).reshape(bsz, od, icaps)
    b_t = b.T
    v_t, s1_t = pl.pallas_call(
        _routing_body,
        grid=(bsz // _G,),
        in_specs=[
            pl.BlockSpec((_G, od, icaps), lambda i: (i, 0, 0)),
            pl.BlockSpec((ocaps, icaps), lambda i: (0, 0)),
        ],
        out_specs=[
            pl.BlockSpec((_G, od, 1), lambda i: (i, 0, 0)),
            pl.BlockSpec((_G, od, icaps), lambda i: (i, 0, 0)),
        ],
        out_shape=[
            jax.ShapeDtypeStruct((bsz, od, 1), jnp.float32),
            jax.ShapeDtypeStruct((bsz, od, icaps), jnp.float32),
        ],
        compiler_params=pltpu.CompilerParams(
            dimension_semantics=("arbitrary",),
        ),
    )(u_t, b_t)
    v = v_t.reshape(bsz, ocaps, dim)
    s1 = s1_t.reshape(bsz, ocaps, dim, icaps).transpose(0, 3, 1, 2)
    return v, s1


# speedup vs baseline: 3.5916x; 1.1787x over previous
import jax
import jax.numpy as jnp
from jax.experimental import pallas as pl
from jax.experimental.pallas import tpu as pltpu

_N_ITER = 5
_G = 4


def _routing_body(u_ref, bt_ref, v_ref, s1_ref):
    g = _G
    ocaps, icaps = bt_ref.shape
    od = u_ref.shape[1]
    dim = od // ocaps
    u = u_ref[...]
    u4 = u.reshape(g, ocaps, dim, icaps)

    b = jnp.broadcast_to(bt_ref[...][None], (g, ocaps, icaps))
    s14 = None
    vcol4 = None
    for r in range(_N_ITER):
        if r > 0:
            y4 = u4 * vcol4
            b = b + jnp.sum(y4, axis=2)
        m = jnp.max(b, axis=1, keepdims=True)
        e = jnp.exp(b - m)
        rs = 1.0 / jnp.sum(e, axis=1, keepdims=True)
        c = e * rs
        s14 = u4 * c[:, :, None, :]
        s24 = jnp.sum(s14, axis=3, keepdims=True)
        n2 = jnp.sum(s24 * s24, axis=2, keepdims=True)
        scale = n2 / (1.0 + n2) / jnp.sqrt(n2)
        vcol4 = s24 * scale
    v_ref[...] = vcol4.reshape(g, od, 1)
    s1_ref[...] = s14.reshape(g, od, icaps)


def kernel(u_predict, b):
    bsz, icaps, ocaps, dim = u_predict.shape
    od = ocaps * dim
    u_t = u_predict.transpose(0, 2, 3, 1).reshape(bsz, od, icaps)
    b_t = b.T
    v_t, s1_t = pl.pallas_call(
        _routing_body,
        grid=(bsz // _G,),
        in_specs=[
            pl.BlockSpec((_G, od, icaps), lambda i: (i, 0, 0)),
            pl.BlockSpec((ocaps, icaps), lambda i: (0, 0)),
        ],
        out_specs=[
            pl.BlockSpec((_G, od, 1), lambda i: (i, 0, 0)),
            pl.BlockSpec((_G, od, icaps), lambda i: (i, 0, 0)),
        ],
        out_shape=[
            jax.ShapeDtypeStruct((bsz, od, 1), jnp.float32),
            jax.ShapeDtypeStruct((bsz, od, icaps), jnp.float32),
        ],
        compiler_params=pltpu.CompilerParams(
            dimension_semantics=("arbitrary",),
        ),
    )(u_t, b_t)
    v = v_t.reshape(bsz, ocaps, dim)
    s1 = s1_t.reshape(bsz, ocaps, dim, icaps).transpose(0, 3, 1, 2)
    return v, s1


# speedup vs baseline: 3.7669x; 1.0488x over previous
import jax
import jax.numpy as jnp
from jax.experimental import pallas as pl
from jax.experimental.pallas import tpu as pltpu

_N_ITER = 5
_G = 8


def _routing_body(u_ref, bt_ref, v_ref, s1_ref):
    g = _G
    ocaps, icaps = bt_ref.shape
    od = u_ref.shape[1]
    dim = od // ocaps
    u = u_ref[...]
    u4 = u.reshape(g, ocaps, dim, icaps)

    b = jnp.broadcast_to(bt_ref[...][None], (g, ocaps, icaps))
    s14 = None
    vcol4 = None
    for r in range(_N_ITER):
        if r > 0:
            y4 = u4 * vcol4
            b = b + jnp.sum(y4, axis=2)
        e = jnp.exp(b)
        rs = 1.0 / jnp.sum(e, axis=1, keepdims=True)
        c = e * rs
        s14 = u4 * c[:, :, None, :]
        s24 = jnp.sum(s14, axis=3, keepdims=True)
        n2 = jnp.sum(s24 * s24, axis=2, keepdims=True)
        scale = n2 / (1.0 + n2) / jnp.sqrt(n2)
        vcol4 = s24 * scale
    v_ref[...] = vcol4.reshape(g, od, 1)
    s1_ref[...] = s14.reshape(g, od, icaps)


def kernel(u_predict, b):
    bsz, icaps, ocaps, dim = u_predict.shape
    od = ocaps * dim
    u_t = u_predict.transpose(0, 2, 3, 1).reshape(bsz, od, icaps)
    b_t = b.T
    v_t, s1_t = pl.pallas_call(
        _routing_body,
        grid=(bsz // _G,),
        in_specs=[
            pl.BlockSpec((_G, od, icaps), lambda i: (i, 0, 0)),
            pl.BlockSpec((ocaps, icaps), lambda i: (0, 0)),
        ],
        out_specs=[
            pl.BlockSpec((_G, od, 1), lambda i: (i, 0, 0)),
            pl.BlockSpec((_G, od, icaps), lambda i: (i, 0, 0)),
        ],
        out_shape=[
            jax.ShapeDtypeStruct((bsz, od, 1), jnp.float32),
            jax.ShapeDtypeStruct((bsz, od, icaps), jnp.float32),
        ],
        compiler_params=pltpu.CompilerParams(
            dimension_semantics=("parallel",),
        ),
    )(u_t, b_t)
    v = v_t.reshape(bsz, ocaps, dim)
    s1 = s1_t.reshape(bsz, ocaps, dim, icaps).transpose(0, 3, 1, 2)
    return v, s1


# speedup vs baseline: 5.8549x; 1.5543x over previous
import jax
import jax.numpy as jnp
from jax.experimental import pallas as pl
from jax.experimental.pallas import tpu as pltpu

_N_ITER = 5
_G = 8


def _routing_body(u_ref, bt_ref, v_ref, s1_ref):
    g = _G
    ocaps, icaps = bt_ref.shape
    od = u_ref.shape[1]
    dim = od // ocaps
    u = u_ref[...]
    u4 = u.reshape(g, ocaps, dim, icaps)
    ub = u.astype(jnp.bfloat16)
    ko = jax.lax.broadcasted_iota(jnp.int32, (ocaps, od), 1) // dim
    oo = jax.lax.broadcasted_iota(jnp.int32, (ocaps, od), 0)
    omask = (ko == oo).astype(jnp.float32)

    b = jnp.broadcast_to(bt_ref[...][None], (g, ocaps, icaps))
    s14 = None
    vcol4 = None
    for r in range(_N_ITER):
        if r > 0:
            vrow = jnp.swapaxes(vcol4.reshape(g, od, 1), 1, 2)
            w = (omask[None] * vrow).astype(jnp.bfloat16)
            z = jax.lax.dot_general(
                w, ub, (((2,), (1,)), ((0,), (0,))),
                preferred_element_type=jnp.float32)
            b = b + z
        e = jnp.exp(b)
        rs = 1.0 / jnp.sum(e, axis=1, keepdims=True)
        c = e * rs
        if r == _N_ITER - 1:
            s14 = u4 * c[:, :, None, :]
            s24 = jnp.sum(s14, axis=3, keepdims=True)
        else:
            s24 = jnp.sum(u4 * c[:, :, None, :], axis=3, keepdims=True)
        n2 = jnp.sum(s24 * s24, axis=2, keepdims=True)
        scale = jnp.sqrt(n2) / (1.0 + n2)
        vcol4 = s24 * scale
    v_ref[...] = vcol4.reshape(g, od, 1)
    s1_ref[...] = s14.reshape(g, od, icaps)


def kernel(u_predict, b):
    bsz, icaps, ocaps, dim = u_predict.shape
    od = ocaps * dim
    u_t = u_predict.transpose(0, 2, 3, 1).reshape(bsz, od, icaps)
    b_t = b.T
    v_t, s1_t = pl.pallas_call(
        _routing_body,
        grid=(bsz // _G,),
        in_specs=[
            pl.BlockSpec((_G, od, icaps), lambda i: (i, 0, 0)),
            pl.BlockSpec((ocaps, icaps), lambda i: (0, 0)),
        ],
        out_specs=[
            pl.BlockSpec((_G, od, 1), lambda i: (i, 0, 0)),
            pl.BlockSpec((_G, od, icaps), lambda i: (i, 0, 0)),
        ],
        out_shape=[
            jax.ShapeDtypeStruct((bsz, od, 1), jnp.float32),
            jax.ShapeDtypeStruct((bsz, od, icaps), jnp.float32),
        ],
        compiler_params=pltpu.CompilerParams(
            dimension_semantics=("parallel",),
        ),
    )(u_t, b_t)
    v = v_t.reshape(bsz, ocaps, dim)
    s1 = s1_t.reshape(bsz, ocaps, dim, icaps).transpose(0, 3, 1, 2)
    return v, s1


# speedup vs baseline: 5.9171x; 1.0106x over previous
import jax
import jax.numpy as jnp
from jax.experimental import pallas as pl
from jax.experimental.pallas import tpu as pltpu

_N_ITER = 5
_G = 16


def _routing_body(u_ref, bt_ref, v_ref, s1_ref):
    g = _G
    ocaps, icaps = bt_ref.shape
    od = u_ref.shape[1]
    dim = od // ocaps
    u = u_ref[...]
    u4 = u.reshape(g, ocaps, dim, icaps)
    ub = u.astype(jnp.bfloat16)
    ko = jax.lax.broadcasted_iota(jnp.int32, (ocaps, od), 1) // dim
    oo = jax.lax.broadcasted_iota(jnp.int32, (ocaps, od), 0)
    omask = (ko == oo).astype(jnp.float32)

    b = jnp.broadcast_to(bt_ref[...][None], (g, ocaps, icaps))
    s14 = None
    vcol4 = None
    for r in range(_N_ITER):
        if r > 0:
            vrow = jnp.swapaxes(vcol4.reshape(g, od, 1), 1, 2)
            w = (omask[None] * vrow).astype(jnp.bfloat16)
            z = jax.lax.dot_general(
                w, ub, (((2,), (1,)), ((0,), (0,))),
                preferred_element_type=jnp.float32)
            b = b + z
        e = jnp.exp(b)
        rs = jax.lax.reciprocal(jnp.sum(e, axis=1, keepdims=True))
        c = e * rs
        if r == _N_ITER - 1:
            s14 = u4 * c[:, :, None, :]
            s24 = jnp.sum(s14, axis=3, keepdims=True)
        else:
            s24 = jnp.sum(u4 * c[:, :, None, :], axis=3, keepdims=True)
        n2 = jnp.sum(s24 * s24, axis=2, keepdims=True)
        scale = jnp.sqrt(n2) * jax.lax.reciprocal(1.0 + n2)
        vcol4 = s24 * scale
    v_ref[...] = vcol4.reshape(g, od, 1)
    s1_ref[...] = s14.reshape(g, od, icaps)


def kernel(u_predict, b):
    bsz, icaps, ocaps, dim = u_predict.shape
    od = ocaps * dim
    u_t = u_predict.transpose(0, 2, 3, 1).reshape(bsz, od, icaps)
    b_t = b.T
    v_t, s1_t = pl.pallas_call(
        _routing_body,
        grid=(bsz // _G,),
        in_specs=[
            pl.BlockSpec((_G, od, icaps), lambda i: (i, 0, 0)),
            pl.BlockSpec((ocaps, icaps), lambda i: (0, 0)),
        ],
        out_specs=[
            pl.BlockSpec((_G, od, 1), lambda i: (i, 0, 0)),
            pl.BlockSpec((_G, od, icaps), lambda i: (i, 0, 0)),
        ],
        out_shape=[
            jax.ShapeDtypeStruct((bsz, od, 1), jnp.float32),
            jax.ShapeDtypeStruct((bsz, od, icaps), jnp.float32),
        ],
        compiler_params=pltpu.CompilerParams(
            dimension_semantics=("parallel",),
        ),
    )(u_t, b_t)
    v = v_t.reshape(bsz, ocaps, dim)
    s1 = s1_t.reshape(bsz, ocaps, dim, icaps).transpose(0, 3, 1, 2)
    return v, s1
